# baseline (speedup 1.0000x reference)
import jax
import jax.numpy as jnp
from jax import lax
from jax.experimental import pallas as pl
from jax.experimental.pallas import tpu as pltpu
from jax.experimental.pallas import tpu_sc as plsc

N_NODES = 10000
N_EDGES = 320000
D = 128

NC = 2
NS = 16
L = 16

N_PAD = 10240
ROWS_PER_TILE = N_PAD // NS
GROUPS_PER_TILE = ROWS_PER_TILE // L
RC = 80
N_RCHUNK = ROWS_PER_TILE // RC

EC = 128
E_PAD = 2560 * EC
NCH = E_PAD // EC
DCH_TILE = NCH // NS
ECH_TILE = NCH // (NC * NS)
SLAB = 40


def _rsqrt16(v):
    i = lax.bitcast_convert_type(v, jnp.int32)
    i = jnp.int32(0x5F3759DF) - lax.shift_right_logical(i, 1)
    y = lax.bitcast_convert_type(i, jnp.float32)
    half = v * 0.5
    for _ in range(3):
        y = y * (1.5 - half * y * y)
    return y


def _scale_rows(vb, dv, base):
    for r in range(RC):
        bc = plsc.load_gather(dv, [jnp.full((L,), base + r, jnp.int32)])
        for j in range(D // L):
            vb[r, pl.ds(j * L, L)] = vb[r, pl.ds(j * L, L)] * bc


def _sc_body(x_hbm, row2d_hbm, col2d_hbm,
             part_hbm, xs_hbm,
             vb0, vb1, rb2d, cb2d,
             ones, dv,
             deg_sh, agg_sh,
             semA, semB):
    c = lax.axis_index("c")
    s = lax.axis_index("s")
    row0 = s * ROWS_PER_TILE

    for g in range(EC // L):
        ones[pl.ds(g * L, L)] = jnp.full((L,), 1.0, jnp.float32)

    def zvb(r, carry):
        for j in range(D // L):
            vb0[r, pl.ds(j * L, L)] = jnp.zeros((L,), jnp.float32)
        return carry
    lax.fori_loop(0, EC, zvb, 0)

    def zdv(k, carry):
        dv[pl.ds(k * L, L)] = jnp.zeros((L,), jnp.float32)
        return carry
    lax.fori_loop(0, GROUPS_PER_TILE, zdv, 0)
    pltpu.sync_copy(dv, deg_sh.at[pl.ds(row0, ROWS_PER_TILE)])
    for g in range(ROWS_PER_TILE // EC):
        pltpu.sync_copy(vb0, agg_sh.at[pl.ds(row0 + g * EC, EC)])

    plsc.subcore_barrier()

    def dsc_start(slab, j, sem):
        return pltpu.async_copy(ones, deg_sh.at[slab.at[j]], sem, add=True)

    def dsc_wait(slab, j, sem):
        pltpu.make_async_copy(ones, deg_sh.at[slab.at[j]], sem).wait()

    for h in range(DCH_TILE // (2 * SLAB)):
        dbase = s * DCH_TILE + h * 2 * SLAB
        pltpu.sync_copy(col2d_hbm.at[pl.ds(dbase, SLAB)], rb2d)
        pltpu.sync_copy(col2d_hbm.at[pl.ds(dbase + SLAB, SLAB)], cb2d)
        dsc_start(rb2d, 0, semA)

        def deg_step(k, carry):
            dsc_start(cb2d, k, semB)
            dsc_wait(rb2d, k, semA)

            @pl.when(k + 1 < SLAB)
            def _():
                dsc_start(rb2d, k + 1, semA)
            dsc_wait(cb2d, k, semB)
            return carry
        lax.fori_loop(0, SLAB, deg_step, 0)
    plsc.subcore_barrier()

    pltpu.sync_copy(deg_sh.at[pl.ds(row0, ROWS_PER_TILE)], dv)

    def dinv_step(k, carry):
        v = dv[pl.ds(k * L, L)]
        y = jnp.where(v >= 0.5, _rsqrt16(v), 0.0)
        dv[pl.ds(k * L, L)] = y
        return carry
    lax.fori_loop(0, GROUPS_PER_TILE, dinv_step, 0)

    def xs_step(kk, carry):
        start = row0 + kk * RC

        @pl.when(start + RC <= N_NODES)
        def _():
            pltpu.sync_copy(x_hbm.at[pl.ds(start, RC)], vb0.at[pl.ds(0, RC)])
            _scale_rows(vb0, dv, kk * RC)
            pltpu.sync_copy(vb0.at[pl.ds(0, RC)], xs_hbm.at[pl.ds(start, RC)])
        return carry
    lax.fori_loop(0, N_RCHUNK, xs_step, 0)
    plsc.subcore_barrier()

    estart = (c * NS + s) * ECH_TILE

    def g_start(j, vb, sem):
        return pltpu.async_copy(xs_hbm.at[rb2d.at[j]], vb, sem)

    def g_wait(j, vb, sem):
        pltpu.make_async_copy(xs_hbm.at[rb2d.at[j]], vb, sem).wait()

    for h in range(ECH_TILE // SLAB):
        ebase = estart + h * SLAB
        pltpu.sync_copy(row2d_hbm.at[pl.ds(ebase, SLAB)], rb2d)
        pltpu.sync_copy(col2d_hbm.at[pl.ds(ebase, SLAB)], cb2d)
        g_start(0, vb0, semA)

        def edge_step(k, carry):
            a = 2 * k
            g_start(a + 1, vb1, semB)
            g_wait(a, vb0, semA)
            pltpu.sync_copy(vb0, agg_sh.at[cb2d.at[a]], add=True)

            @pl.when(a + 2 < SLAB)
            def _():
                g_start(a + 2, vb0, semA)
            g_wait(a + 1, vb1, semB)
            pltpu.sync_copy(vb1, agg_sh.at[cb2d.at[a + 1]], add=True)
            return carry
        lax.fori_loop(0, SLAB // 2, edge_step, 0)
    plsc.subcore_barrier()

    def out_step(kk, carry):
        start = row0 + kk * RC
        pltpu.sync_copy(agg_sh.at[pl.ds(start, RC)], vb0.at[pl.ds(0, RC)])
        _scale_rows(vb0, dv, kk * RC)
        pltpu.sync_copy(vb0.at[pl.ds(0, RC)], part_hbm.at[c, pl.ds(start, RC)])
        return carry
    lax.fori_loop(0, N_RCHUNK, out_step, 0)


_sc_kernel = pl.kernel(
    _sc_body,
    out_type=[
        jax.ShapeDtypeStruct((NC, N_PAD, D), jnp.float32),
        jax.ShapeDtypeStruct((N_NODES, D), jnp.float32),
    ],
    mesh=plsc.VectorSubcoreMesh(core_axis_name="c", subcore_axis_name="s"),
    compiler_params=pltpu.CompilerParams(needs_layout_passes=False),
    scratch_types=[
        pltpu.VMEM((EC, D), jnp.float32),
        pltpu.VMEM((EC, D), jnp.float32),
        pltpu.VMEM((SLAB, EC), jnp.int32),
        pltpu.VMEM((SLAB, EC), jnp.int32),
        pltpu.VMEM((EC,), jnp.float32),
        pltpu.VMEM((ROWS_PER_TILE,), jnp.float32),
        pltpu.VMEM_SHARED((N_PAD,), jnp.float32),
        pltpu.VMEM_SHARED((N_PAD, D), jnp.float32),
        pltpu.SemaphoreType.DMA,
        pltpu.SemaphoreType.DMA,
    ],
)


def _mm_body(p_ref, wt_ref, b_ref, o_ref):
    a = p_ref[0] + p_ref[1]
    o_ref[...] = (
        jnp.dot(a, wt_ref[...], preferred_element_type=jnp.float32) + b_ref[...]
    )


_BM = 512


def _matmul(parts, wt, b2):
    return pl.pallas_call(
        _mm_body,
        grid=(N_PAD // _BM,),
        in_specs=[
            pl.BlockSpec((NC, _BM, D), lambda i: (0, i, 0)),
            pl.BlockSpec((D, D), lambda i: (0, 0)),
            pl.BlockSpec((1, D), lambda i: (0, 0)),
        ],
        out_specs=pl.BlockSpec((_BM, D), lambda i: (i, 0)),
        out_shape=jax.ShapeDtypeStruct((N_PAD, D), jnp.float32),
    )(parts, wt, b2)


def kernel(x, edge_index, x0, W, b):
    row = edge_index[0].astype(jnp.int32)
    col = edge_index[1].astype(jnp.int32)
    n_fill = E_PAD - row.shape[0]
    row2d = jnp.concatenate(
        [row, jnp.zeros((n_fill,), jnp.int32)]).reshape(NCH, EC)
    col2d = jnp.concatenate(
        [col, jnp.full((n_fill,), N_PAD - 1, jnp.int32)]).reshape(NCH, EC)
    parts, _ = _sc_kernel(x, row2d, col2d)
    out = _matmul(parts, W.T, b.reshape(1, D))
    return out[:N_NODES]


# speedup vs baseline: 3.0602x; 3.0602x over previous
import jax
import jax.numpy as jnp
from jax import lax
from jax.experimental import pallas as pl
from jax.experimental.pallas import tpu as pltpu
from jax.experimental.pallas import tpu_sc as plsc

N_NODES = 10000
N_EDGES = 320000
D = 128

NC = 2
NS = 16
L = 16

N_PAD = 10240
ROWS_PER_TILE = N_PAD // NS
GROUPS_PER_TILE = ROWS_PER_TILE // L
RC = 80
N_RCHUNK = ROWS_PER_TILE // RC

EC = 128
NCH = N_EDGES // EC
SLAB = 40
N_SLAB = NCH // SLAB
TAIL = NCH - N_SLAB * SLAB


def _rsqrt16(v):
    i = lax.bitcast_convert_type(v, jnp.int32)
    i = jnp.int32(0x5F3759DF) - lax.shift_right_logical(i, 1)
    y = lax.bitcast_convert_type(i, jnp.float32)
    half = v * 0.5
    for _ in range(3):
        y = y * (1.5 - half * y * y)
    return y


def _scale_rows(vb, dv, base):
    for r in range(RC):
        bc = plsc.load_gather(dv, [jnp.full((L,), base + r, jnp.int32)])
        for j in range(D // L):
            vb[r, pl.ds(j * L, L)] = vb[r, pl.ds(j * L, L)] * bc


def _sc_body(x_hbm, e3d_hbm,
             part_hbm, xs_hbm, dinv_hbm,
             vb0, vb1, rb2d, cb2d,
             ones, dv,
             deg_sh, agg_sh,
             semA, semB):
    c = lax.axis_index("c")
    s = lax.axis_index("s")
    row0 = s * ROWS_PER_TILE

    with jax.named_scope("p0_init"):
        for g in range(EC // L):
            ones[pl.ds(g * L, L)] = jnp.full((L,), 1.0, jnp.float32)

        def zvb(r, carry):
            for j in range(D // L):
                vb0[r, pl.ds(j * L, L)] = jnp.zeros((L,), jnp.float32)
            return carry
        lax.fori_loop(0, EC, zvb, 0)

        def zdv(k, carry):
            dv[pl.ds(k * L, L)] = jnp.zeros((L,), jnp.float32)
            return carry
        lax.fori_loop(0, GROUPS_PER_TILE, zdv, 0)
        pltpu.sync_copy(dv, deg_sh.at[pl.ds(row0, ROWS_PER_TILE)])
        for g in range(ROWS_PER_TILE // EC):
            pltpu.sync_copy(vb0, agg_sh.at[pl.ds(row0 + g * EC, EC)])
        plsc.subcore_barrier()

    with jax.named_scope("p1_degree"):
        def hist_run(chunk0, n):
            pltpu.sync_copy(e3d_hbm.at[1, pl.ds(chunk0, n)],
                            rb2d.at[pl.ds(0, n)])
            pltpu.async_copy(ones, deg_sh.at[rb2d.at[0]], semA, add=True)

            def step(k, carry):
                a = 2 * k
                pltpu.async_copy(ones, deg_sh.at[rb2d.at[a + 1]], semB,
                                 add=True)
                pltpu.make_async_copy(ones, deg_sh.at[rb2d.at[a]],
                                      semA).wait()

                @pl.when(a + 2 < n)
                def _():
                    pltpu.async_copy(ones, deg_sh.at[rb2d.at[a + 2]], semA,
                                     add=True)
                pltpu.make_async_copy(ones, deg_sh.at[rb2d.at[a + 1]],
                                      semB).wait()
                return carry
            lax.fori_loop(0, n // 2, step, 0)

        for hh in range(4):
            @pl.when(s + 16 * hh < N_SLAB)
            def _():
                hist_run((s + 16 * hh) * SLAB, SLAB)

        @pl.when(s == NS - 1)
        def _():
            hist_run(N_SLAB * SLAB, TAIL)
        plsc.subcore_barrier()

    with jax.named_scope("p2_dinv"):
        pltpu.sync_copy(deg_sh.at[pl.ds(row0, ROWS_PER_TILE)], dv)

        def dinv_step(k, carry):
            v = dv[pl.ds(k * L, L)]
            y = jnp.where(v >= 0.5, _rsqrt16(v), 0.0)
            dv[pl.ds(k * L, L)] = y
            return carry
        lax.fori_loop(0, GROUPS_PER_TILE, dinv_step, 0)

        @pl.when(c == 0)
        def _():
            pltpu.sync_copy(dv, dinv_hbm.at[pl.ds(row0, ROWS_PER_TILE)])

    with jax.named_scope("p3_xs"):
        def p3_valid(kk):
            return row0 + kk * RC + RC <= N_NODES

        def p3_load(kk, vb, sem):
            return pltpu.async_copy(
                x_hbm.at[pl.ds(row0 + kk * RC, RC)], vb.at[pl.ds(0, RC)], sem)

        def p3_do(kk, vb, sem):
            @pl.when(p3_valid(kk))
            def _():
                pltpu.make_async_copy(
                    x_hbm.at[pl.ds(row0 + kk * RC, RC)],
                    vb.at[pl.ds(0, RC)], sem).wait()
                _scale_rows(vb, dv, kk * RC)
                pltpu.sync_copy(vb.at[pl.ds(0, RC)],
                                xs_hbm.at[pl.ds(row0 + kk * RC, RC)])

        @pl.when(p3_valid(0))
        def _():
            p3_load(0, vb0, semA)

        def p3_step(k2, carry):
            a = 2 * k2

            @pl.when(p3_valid(a + 1))
            def _():
                p3_load(a + 1, vb1, semB)
            p3_do(a, vb0, semA)

            @pl.when((a + 2 < N_RCHUNK) & p3_valid(a + 2))
            def _():
                p3_load(a + 2, vb0, semA)
            p3_do(a + 1, vb1, semB)
            return carry
        lax.fori_loop(0, N_RCHUNK // 2, p3_step, 0)
        plsc.subcore_barrier()

    with jax.named_scope("p4_edges"):
        def g_start(j, vb, sem):
            return pltpu.async_copy(xs_hbm.at[rb2d.at[j]], vb, sem)

        def g_wait(j, vb, sem):
            pltpu.make_async_copy(xs_hbm.at[rb2d.at[j]], vb, sem).wait()

        def edge_run(chunk0, n):
            pltpu.sync_copy(e3d_hbm.at[0, pl.ds(chunk0, n)],
                            rb2d.at[pl.ds(0, n)])
            pltpu.sync_copy(e3d_hbm.at[1, pl.ds(chunk0, n)],
                            cb2d.at[pl.ds(0, n)])
            g_start(0, vb0, semA)

            def step(k, carry):
                a = 2 * k
                g_start(a + 1, vb1, semB)
                g_wait(a, vb0, semA)
                pltpu.sync_copy(vb0, agg_sh.at[cb2d.at[a]], add=True)

                @pl.when(a + 2 < n)
                def _():
                    g_start(a + 2, vb0, semA)
                g_wait(a + 1, vb1, semB)
                pltpu.sync_copy(vb1, agg_sh.at[cb2d.at[a + 1]], add=True)
                return carry
            lax.fori_loop(0, n // 2, step, 0)

        for hh in range(2):
            @pl.when(s + 16 * hh < N_SLAB // 2)
            def _():
                edge_run((2 * (s + 16 * hh) + c) * SLAB, SLAB)

        @pl.when((s == NS - 1) & (c == 0))
        def _():
            edge_run(N_SLAB * SLAB, TAIL)
        plsc.subcore_barrier()

    with jax.named_scope("p5_out"):
        for g in range(ROWS_PER_TILE // EC):
            vb = vb0 if g % 2 == 0 else vb1
            sem = semA if g % 2 == 0 else semB
            if g >= 2:
                pltpu.make_async_copy(
                    vb, part_hbm.at[c, pl.ds(row0 + (g - 2) * EC, EC)],
                    sem).wait()
            pltpu.sync_copy(agg_sh.at[pl.ds(row0 + g * EC, EC)], vb)
            pltpu.async_copy(vb, part_hbm.at[c, pl.ds(row0 + g * EC, EC)],
                             sem)
        for g in (3, 4):
            vb = vb0 if g % 2 == 0 else vb1
            sem = semA if g % 2 == 0 else semB
            pltpu.make_async_copy(
                vb, part_hbm.at[c, pl.ds(row0 + g * EC, EC)], sem).wait()


_sc_kernel = pl.kernel(
    _sc_body,
    out_type=[
        jax.ShapeDtypeStruct((NC, N_PAD, D), jnp.float32),
        jax.ShapeDtypeStruct((N_NODES, D), jnp.float32),
        jax.ShapeDtypeStruct((N_PAD,), jnp.float32),
    ],
    mesh=plsc.VectorSubcoreMesh(core_axis_name="c", subcore_axis_name="s"),
    compiler_params=pltpu.CompilerParams(needs_layout_passes=False),
    scratch_types=[
        pltpu.VMEM((EC, D), jnp.float32),
        pltpu.VMEM((EC, D), jnp.float32),
        pltpu.VMEM((SLAB, EC), jnp.int32),
        pltpu.VMEM((SLAB, EC), jnp.int32),
        pltpu.VMEM((EC,), jnp.float32),
        pltpu.VMEM((ROWS_PER_TILE,), jnp.float32),
        pltpu.VMEM_SHARED((N_PAD,), jnp.float32),
        pltpu.VMEM_SHARED((N_PAD, D), jnp.float32),
        pltpu.SemaphoreType.DMA,
        pltpu.SemaphoreType.DMA,
    ],
)


def _mm_body(p_ref, dv_ref, wt_ref, b_ref, o_ref):
    a = (p_ref[0] + p_ref[1]) * dv_ref[0][:, :, None]
    a = a.reshape(a.shape[0] * a.shape[1], a.shape[2])
    o_ref[...] = (
        jnp.dot(a, wt_ref[...], preferred_element_type=jnp.float32) + b_ref[...]
    )


_BM = 512
_BG = _BM // 128


def _matmul(parts4d, dinv2d, wt, b2):
    return pl.pallas_call(
        _mm_body,
        grid=(N_PAD // _BM,),
        in_specs=[
            pl.BlockSpec((NC, _BG, 128, D), lambda i: (0, i, 0, 0)),
            pl.BlockSpec((1, _BG, 128), lambda i: (i, 0, 0)),
            pl.BlockSpec((D, D), lambda i: (0, 0)),
            pl.BlockSpec((1, D), lambda i: (0, 0)),
        ],
        out_specs=pl.BlockSpec((_BM, D), lambda i: (i, 0)),
        out_shape=jax.ShapeDtypeStruct((N_PAD, D), jnp.float32),
    )(parts4d, dinv2d, wt, b2)


def kernel(x, edge_index, x0, W, b):
    e3d = edge_index.astype(jnp.int32).reshape(NC, NCH, EC)
    parts, _, dinv = _sc_kernel(x, e3d)
    out = _matmul(parts.reshape(NC, N_PAD // 128, 128, D),
                  dinv.reshape(N_PAD // _BM, _BG, 128), W.T, b.reshape(1, D))
    return out[:N_NODES]


# speedup vs baseline: 3.3363x; 1.0902x over previous
import jax
import jax.numpy as jnp
from jax import lax
from jax.experimental import pallas as pl
from jax.experimental.pallas import tpu as pltpu
from jax.experimental.pallas import tpu_sc as plsc

N_NODES = 10000
N_EDGES = 320000
D = 128

NC = 2
NS = 16
L = 16

N_PAD = 10240
ROWS_PER_TILE = N_PAD // NS
GROUPS_PER_TILE = ROWS_PER_TILE // L
RC = 80
N_RCHUNK = ROWS_PER_TILE // RC

EC = 128
NCH = N_EDGES // EC
SLAB = 40
N_SLAB = NCH // SLAB
TAIL = NCH - N_SLAB * SLAB


def _rsqrt16(v):
    i = lax.bitcast_convert_type(v, jnp.int32)
    i = jnp.int32(0x5F3759DF) - lax.shift_right_logical(i, 1)
    y = lax.bitcast_convert_type(i, jnp.float32)
    half = v * 0.5
    for _ in range(3):
        y = y * (1.5 - half * y * y)
    return y


def _scale_rows(vb, dv, base):
    for g in range(RC // L):
        dvv = dv[pl.ds(base + g * L, L)]
        for r in range(L):
            bc = lax.gather(
                dvv, jnp.full((L, 1), r, jnp.int32),
                lax.GatherDimensionNumbers(offset_dims=(),
                                           collapsed_slice_dims=(0,),
                                           start_index_map=(0,)),
                slice_sizes=(1,),
                mode=lax.GatherScatterMode.PROMISE_IN_BOUNDS)
            row = g * L + r
            for j in range(D // L):
                vb[row, pl.ds(j * L, L)] = vb[row, pl.ds(j * L, L)] * bc


def _sc_body(x_hbm, e3d_hbm,
             part_hbm, xs_hbm, dinv_hbm,
             vb0, vb1, rb2d, cb2d,
             ones, dv,
             deg_sh, agg_sh,
             semA, semB, semC, semD):
    c = lax.axis_index("c")
    s = lax.axis_index("s")
    row0 = s * ROWS_PER_TILE

    with jax.named_scope("p0_init"):
        for g in range(EC // L):
            ones[pl.ds(g * L, L)] = jnp.full((L,), 1.0, jnp.float32)

        def zvb(r, carry):
            for j in range(D // L):
                vb0[r, pl.ds(j * L, L)] = jnp.zeros((L,), jnp.float32)
            return carry
        lax.fori_loop(0, EC, zvb, 0)

        def zdv(k, carry):
            dv[pl.ds(k * L, L)] = jnp.zeros((L,), jnp.float32)
            return carry
        lax.fori_loop(0, GROUPS_PER_TILE, zdv, 0)
        pltpu.sync_copy(dv, deg_sh.at[pl.ds(row0, ROWS_PER_TILE)])
        for g in range(ROWS_PER_TILE // EC):
            pltpu.sync_copy(vb0, agg_sh.at[pl.ds(row0 + g * EC, EC)])
        plsc.subcore_barrier()

    with jax.named_scope("p1_degree"):
        sems = (semA, semB, semC, semD)

        def dsc_start(j, sem):
            pltpu.async_copy(ones, deg_sh.at[rb2d.at[j]], sem, add=True)

        def dsc_wait(j, sem):
            pltpu.make_async_copy(ones, deg_sh.at[rb2d.at[j]], sem).wait()

        def hist_run(chunk0, n):
            pltpu.sync_copy(e3d_hbm.at[1, pl.ds(chunk0, n)],
                            rb2d.at[pl.ds(0, n)])
            for j in range(4):
                dsc_start(j, sems[j])

            def step(k, carry):
                a = 4 * k
                for q in range(4):
                    dsc_wait(a + q, sems[q])

                    @pl.when(a + 4 + q < n)
                    def _(q=q):
                        dsc_start(a + 4 + q, sems[q])
                return carry
            lax.fori_loop(0, n // 4, step, 0)

        for hh in range(4):
            @pl.when(s + 16 * hh < N_SLAB)
            def _():
                hist_run((s + 16 * hh) * SLAB, SLAB)

        @pl.when(s == NS - 1)
        def _():
            hist_run(N_SLAB * SLAB, TAIL)
        plsc.subcore_barrier()

    with jax.named_scope("p2_dinv"):
        pltpu.sync_copy(deg_sh.at[pl.ds(row0, ROWS_PER_TILE)], dv)

        def dinv_step(k, carry):
            v = dv[pl.ds(k * L, L)]
            y = jnp.where(v >= 0.5, _rsqrt16(v), 0.0)
            dv[pl.ds(k * L, L)] = y
            return carry
        lax.fori_loop(0, GROUPS_PER_TILE, dinv_step, 0)

        @pl.when(c == 0)
        def _():
            pltpu.sync_copy(dv, dinv_hbm.at[pl.ds(row0, ROWS_PER_TILE)])

    with jax.named_scope("p3_xs"):
        def p3_valid(kk):
            return row0 + kk * RC + RC <= N_NODES

        def p3_load(kk, vb, sem):
            return pltpu.async_copy(
                x_hbm.at[pl.ds(row0 + kk * RC, RC)], vb.at[pl.ds(0, RC)], sem)

        def p3_do(kk, vb, sem):
            @pl.when(p3_valid(kk))
            def _():
                pltpu.make_async_copy(
                    x_hbm.at[pl.ds(row0 + kk * RC, RC)],
                    vb.at[pl.ds(0, RC)], sem).wait()
                _scale_rows(vb, dv, kk * RC)
                pltpu.sync_copy(vb.at[pl.ds(0, RC)],
                                xs_hbm.at[pl.ds(row0 + kk * RC, RC)])

        @pl.when(p3_valid(0))
        def _():
            p3_load(0, vb0, semA)

        def p3_step(k2, carry):
            a = 2 * k2

            @pl.when(p3_valid(a + 1))
            def _():
                p3_load(a + 1, vb1, semB)
            p3_do(a, vb0, semA)

            @pl.when((a + 2 < N_RCHUNK) & p3_valid(a + 2))
            def _():
                p3_load(a + 2, vb0, semA)
            p3_do(a + 1, vb1, semB)
            return carry
        lax.fori_loop(0, N_RCHUNK // 2, p3_step, 0)
        plsc.subcore_barrier()

    with jax.named_scope("p4_edges"):
        def g_start(j, vb, sem):
            return pltpu.async_copy(xs_hbm.at[rb2d.at[j]], vb, sem)

        def g_wait(j, vb, sem):
            pltpu.make_async_copy(xs_hbm.at[rb2d.at[j]], vb, sem).wait()

        def edge_run(chunk0, n):
            pltpu.sync_copy(e3d_hbm.at[0, pl.ds(chunk0, n)],
                            rb2d.at[pl.ds(0, n)])
            pltpu.sync_copy(e3d_hbm.at[1, pl.ds(chunk0, n)],
                            cb2d.at[pl.ds(0, n)])
            g_start(0, vb0, semA)

            def step(k, carry):
                a = 2 * k
                g_start(a + 1, vb1, semB)
                g_wait(a, vb0, semA)
                pltpu.sync_copy(vb0, agg_sh.at[cb2d.at[a]], add=True)

                @pl.when(a + 2 < n)
                def _():
                    g_start(a + 2, vb0, semA)
                g_wait(a + 1, vb1, semB)
                pltpu.sync_copy(vb1, agg_sh.at[cb2d.at[a + 1]], add=True)
                return carry
            lax.fori_loop(0, n // 2, step, 0)

        for hh in range(2):
            @pl.when(s + 16 * hh < N_SLAB // 2)
            def _():
                edge_run((2 * (s + 16 * hh) + c) * SLAB, SLAB)

        @pl.when((s == NS - 1) & (c == 0))
        def _():
            edge_run(N_SLAB * SLAB, TAIL)
        plsc.subcore_barrier()

    with jax.named_scope("p5_out"):
        for g in range(ROWS_PER_TILE // EC):
            vb = vb0 if g % 2 == 0 else vb1
            sem = semA if g % 2 == 0 else semB
            if g >= 2:
                pltpu.make_async_copy(
                    vb, part_hbm.at[c, pl.ds(row0 + (g - 2) * EC, EC)],
                    sem).wait()
            pltpu.sync_copy(agg_sh.at[pl.ds(row0 + g * EC, EC)], vb)
            pltpu.async_copy(vb, part_hbm.at[c, pl.ds(row0 + g * EC, EC)],
                             sem)
        for g in (3, 4):
            vb = vb0 if g % 2 == 0 else vb1
            sem = semA if g % 2 == 0 else semB
            pltpu.make_async_copy(
                vb, part_hbm.at[c, pl.ds(row0 + g * EC, EC)], sem).wait()


_sc_kernel = pl.kernel(
    _sc_body,
    out_type=[
        jax.ShapeDtypeStruct((NC, N_PAD, D), jnp.float32),
        jax.ShapeDtypeStruct((N_NODES, D), jnp.float32),
        jax.ShapeDtypeStruct((N_PAD,), jnp.float32),
    ],
    mesh=plsc.VectorSubcoreMesh(core_axis_name="c", subcore_axis_name="s"),
    compiler_params=pltpu.CompilerParams(needs_layout_passes=False),
    scratch_types=[
        pltpu.VMEM((EC, D), jnp.float32),
        pltpu.VMEM((EC, D), jnp.float32),
        pltpu.VMEM((SLAB, EC), jnp.int32),
        pltpu.VMEM((SLAB, EC), jnp.int32),
        pltpu.VMEM((EC,), jnp.float32),
        pltpu.VMEM((ROWS_PER_TILE,), jnp.float32),
        pltpu.VMEM_SHARED((N_PAD,), jnp.float32),
        pltpu.VMEM_SHARED((N_PAD, D), jnp.float32),
        pltpu.SemaphoreType.DMA,
        pltpu.SemaphoreType.DMA,
        pltpu.SemaphoreType.DMA,
        pltpu.SemaphoreType.DMA,
    ],
)


def _mm_body(p_ref, dv_ref, wt_ref, b_ref, o_ref):
    a = (p_ref[0] + p_ref[1]) * dv_ref[0][:, :, None]
    a = a.reshape(a.shape[0] * a.shape[1], a.shape[2])
    o_ref[...] = (
        jnp.dot(a, wt_ref[...], preferred_element_type=jnp.float32) + b_ref[...]
    )


_BM = 1024
_BG = _BM // 128


def _matmul(parts4d, dinv2d, wt, b2):
    return pl.pallas_call(
        _mm_body,
        grid=(N_PAD // _BM,),
        in_specs=[
            pl.BlockSpec((NC, _BG, 128, D), lambda i: (0, i, 0, 0)),
            pl.BlockSpec((1, _BG, 128), lambda i: (i, 0, 0)),
            pl.BlockSpec((D, D), lambda i: (0, 0)),
            pl.BlockSpec((1, D), lambda i: (0, 0)),
        ],
        out_specs=pl.BlockSpec((_BM, D), lambda i: (i, 0)),
        out_shape=jax.ShapeDtypeStruct((N_PAD, D), jnp.float32),
    )(parts4d, dinv2d, wt, b2)


def kernel(x, edge_index, x0, W, b):
    e3d = edge_index.astype(jnp.int32).reshape(NC, NCH, EC)
    parts, _, dinv = _sc_kernel(x, e3d)
    out = _matmul(parts.reshape(NC, N_PAD // 128, 128, D),
                  dinv.reshape(N_PAD // _BM, _BG, 128), W.T, b.reshape(1, D))
    return out[:N_NODES]


# speedup vs baseline: 3.3571x; 1.0062x over previous
import jax
import jax.numpy as jnp
from jax import lax
from jax.experimental import pallas as pl
from jax.experimental.pallas import tpu as pltpu
from jax.experimental.pallas import tpu_sc as plsc

N_NODES = 10000
N_EDGES = 320000
D = 128

NC = 2
NS = 16
L = 16

N_PAD = 10240
ROWS_PER_TILE = N_PAD // NS
GROUPS_PER_TILE = ROWS_PER_TILE // L
RC = 80
N_RCHUNK = ROWS_PER_TILE // RC

EC = 128
NCH = N_EDGES // EC
SLAB = 40
N_SLAB = NCH // SLAB
TAIL = NCH - N_SLAB * SLAB


def _rsqrt16(v):
    i = lax.bitcast_convert_type(v, jnp.int32)
    i = jnp.int32(0x5F3759DF) - lax.shift_right_logical(i, 1)
    y = lax.bitcast_convert_type(i, jnp.float32)
    half = v * 0.5
    for _ in range(3):
        y = y * (1.5 - half * y * y)
    return y


def _scale_rows(vb, dv, base):
    for g in range(RC // L):
        dvv = dv[pl.ds(base + g * L, L)]
        for r in range(L):
            bc = lax.gather(
                dvv, jnp.full((L, 1), r, jnp.int32),
                lax.GatherDimensionNumbers(offset_dims=(),
                                           collapsed_slice_dims=(0,),
                                           start_index_map=(0,)),
                slice_sizes=(1,),
                mode=lax.GatherScatterMode.PROMISE_IN_BOUNDS)
            row = g * L + r
            for j in range(D // L):
                vb[row, pl.ds(j * L, L)] = vb[row, pl.ds(j * L, L)] * bc


def _sc_body(x_hbm, e3d_hbm,
             part_hbm, xs_hbm, dinv_hbm,
             vb0, vb1, rb2d, cb2d,
             ones, dv,
             deg_sh, agg_sh,
             semA, semB, semC, semD, semE, semF):
    c = lax.axis_index("c")
    s = lax.axis_index("s")
    row0 = s * ROWS_PER_TILE

    def p3_valid(kk):
        return row0 + kk * RC + RC <= N_NODES

    def p3_load(kk, vb, sem):
        return pltpu.async_copy(
            x_hbm.at[pl.ds(row0 + kk * RC, RC)], vb.at[pl.ds(0, RC)], sem)

    with jax.named_scope("p0_init"):
        for g in range(EC // L):
            ones[pl.ds(g * L, L)] = jnp.full((L,), 1.0, jnp.float32)

        def zvb(r, carry):
            for j in range(D // L):
                vb0[r, pl.ds(j * L, L)] = jnp.zeros((L,), jnp.float32)
            return carry
        lax.fori_loop(0, EC, zvb, 0)

        def zdv(k, carry):
            dv[pl.ds(k * L, L)] = jnp.zeros((L,), jnp.float32)
            return carry
        lax.fori_loop(0, GROUPS_PER_TILE, zdv, 0)
        pltpu.sync_copy(dv, deg_sh.at[pl.ds(row0, ROWS_PER_TILE)])
        for g in range(ROWS_PER_TILE // EC):
            pltpu.sync_copy(vb0, agg_sh.at[pl.ds(row0 + g * EC, EC)])

        @pl.when(p3_valid(0))
        def _():
            p3_load(0, vb0, semE)

        @pl.when(p3_valid(1))
        def _():
            p3_load(1, vb1, semF)
        plsc.subcore_barrier()

    with jax.named_scope("p1_degree"):
        sems = (semA, semB, semC, semD)

        def dsc_start(j, sem):
            pltpu.async_copy(ones, deg_sh.at[rb2d.at[j]], sem, add=True)

        def dsc_wait(j, sem):
            pltpu.make_async_copy(ones, deg_sh.at[rb2d.at[j]], sem).wait()

        def hist_run(chunk0, n):
            pltpu.sync_copy(e3d_hbm.at[1, pl.ds(chunk0, n)],
                            rb2d.at[pl.ds(0, n)])
            for j in range(4):
                dsc_start(j, sems[j])

            def step(k, carry):
                a = 4 * k
                for q in range(4):
                    dsc_wait(a + q, sems[q])

                    @pl.when(a + 4 + q < n)
                    def _(q=q):
                        dsc_start(a + 4 + q, sems[q])
                return carry
            lax.fori_loop(0, n // 4, step, 0)

        for hh in range(4):
            @pl.when(s + 16 * hh < N_SLAB)
            def _():
                hist_run((s + 16 * hh) * SLAB, SLAB)

        @pl.when(s == NS - 1)
        def _():
            hist_run(N_SLAB * SLAB, TAIL)
        plsc.subcore_barrier()

    with jax.named_scope("p2_dinv"):
        pltpu.sync_copy(deg_sh.at[pl.ds(row0, ROWS_PER_TILE)], dv)

        def dinv_step(k, carry):
            v = dv[pl.ds(k * L, L)]
            y = jnp.where(v >= 0.5, _rsqrt16(v), 0.0)
            dv[pl.ds(k * L, L)] = y
            return carry
        lax.fori_loop(0, GROUPS_PER_TILE, dinv_step, 0)

        @pl.when(c == 0)
        def _():
            pltpu.sync_copy(dv, dinv_hbm.at[pl.ds(row0, ROWS_PER_TILE)])

    with jax.named_scope("p3_xs"):
        def p3_do(kk, vb, sem):
            @pl.when(p3_valid(kk))
            def _():
                pltpu.make_async_copy(
                    x_hbm.at[pl.ds(row0 + kk * RC, RC)],
                    vb.at[pl.ds(0, RC)], sem).wait()
                _scale_rows(vb, dv, kk * RC)
                pltpu.sync_copy(vb.at[pl.ds(0, RC)],
                                xs_hbm.at[pl.ds(row0 + kk * RC, RC)])

        def p3_step(k2, carry):
            a = 2 * k2
            p3_do(a, vb0, semE)

            @pl.when((a + 2 < N_RCHUNK) & p3_valid(a + 2))
            def _():
                p3_load(a + 2, vb0, semE)
            p3_do(a + 1, vb1, semF)

            @pl.when((a + 3 < N_RCHUNK) & p3_valid(a + 3))
            def _():
                p3_load(a + 3, vb1, semF)
            return carry
        lax.fori_loop(0, N_RCHUNK // 2, p3_step, 0)
        plsc.subcore_barrier()

    with jax.named_scope("p4_edges"):
        def g_start(j, vb, sem):
            return pltpu.async_copy(xs_hbm.at[rb2d.at[j]], vb, sem)

        def g_wait(j, vb, sem):
            pltpu.make_async_copy(xs_hbm.at[rb2d.at[j]], vb, sem).wait()

        def edge_run(chunk0, n):
            pltpu.sync_copy(e3d_hbm.at[0, pl.ds(chunk0, n)],
                            rb2d.at[pl.ds(0, n)])
            pltpu.sync_copy(e3d_hbm.at[1, pl.ds(chunk0, n)],
                            cb2d.at[pl.ds(0, n)])
            g_start(0, vb0, semA)

            def step(k, carry):
                a = 2 * k
                g_start(a + 1, vb1, semB)
                g_wait(a, vb0, semA)
                pltpu.sync_copy(vb0, agg_sh.at[cb2d.at[a]], add=True)

                @pl.when(a + 2 < n)
                def _():
                    g_start(a + 2, vb0, semA)
                g_wait(a + 1, vb1, semB)
                pltpu.sync_copy(vb1, agg_sh.at[cb2d.at[a + 1]], add=True)
                return carry
            lax.fori_loop(0, n // 2, step, 0)

        for hh in range(2):
            @pl.when(s + 16 * hh < N_SLAB // 2)
            def _():
                edge_run((2 * (s + 16 * hh) + c) * SLAB, SLAB)

        @pl.when((s == NS - 1) & (c == 0))
        def _():
            edge_run(N_SLAB * SLAB, TAIL)
        plsc.subcore_barrier()

    with jax.named_scope("p5_out"):
        for g in range(ROWS_PER_TILE // EC):
            vb = vb0 if g % 2 == 0 else vb1
            sem = semA if g % 2 == 0 else semB
            if g >= 2:
                pltpu.make_async_copy(
                    vb, part_hbm.at[c, pl.ds(row0 + (g - 2) * EC, EC)],
                    sem).wait()
            pltpu.sync_copy(agg_sh.at[pl.ds(row0 + g * EC, EC)], vb)
            pltpu.async_copy(vb, part_hbm.at[c, pl.ds(row0 + g * EC, EC)],
                             sem)
        for g in (3, 4):
            vb = vb0 if g % 2 == 0 else vb1
            sem = semA if g % 2 == 0 else semB
            pltpu.make_async_copy(
                vb, part_hbm.at[c, pl.ds(row0 + g * EC, EC)], sem).wait()


_sc_kernel = pl.kernel(
    _sc_body,
    out_type=[
        jax.ShapeDtypeStruct((NC, N_PAD, D), jnp.float32),
        jax.ShapeDtypeStruct((N_NODES, D), jnp.float32),
        jax.ShapeDtypeStruct((N_PAD,), jnp.float32),
    ],
    mesh=plsc.VectorSubcoreMesh(core_axis_name="c", subcore_axis_name="s"),
    compiler_params=pltpu.CompilerParams(needs_layout_passes=False),
    scratch_types=[
        pltpu.VMEM((EC, D), jnp.float32),
        pltpu.VMEM((EC, D), jnp.float32),
        pltpu.VMEM((SLAB, EC), jnp.int32),
        pltpu.VMEM((SLAB, EC), jnp.int32),
        pltpu.VMEM((EC,), jnp.float32),
        pltpu.VMEM((ROWS_PER_TILE,), jnp.float32),
        pltpu.VMEM_SHARED((N_PAD,), jnp.float32),
        pltpu.VMEM_SHARED((N_PAD, D), jnp.float32),
        pltpu.SemaphoreType.DMA,
        pltpu.SemaphoreType.DMA,
        pltpu.SemaphoreType.DMA,
        pltpu.SemaphoreType.DMA,
        pltpu.SemaphoreType.DMA,
        pltpu.SemaphoreType.DMA,
    ],
)


def _mm_body(p_ref, dv_ref, wt_ref, b_ref, o_ref):
    a = (p_ref[0] + p_ref[1]) * dv_ref[0][:, :, None]
    a = a.reshape(a.shape[0] * a.shape[1], a.shape[2])
    o_ref[...] = (
        jnp.dot(a, wt_ref[...], preferred_element_type=jnp.float32) + b_ref[...]
    )


_BM = 1024
_BG = _BM // 128


def _matmul(parts4d, dinv2d, wt, b2):
    return pl.pallas_call(
        _mm_body,
        grid=(N_PAD // _BM,),
        in_specs=[
            pl.BlockSpec((NC, _BG, 128, D), lambda i: (0, i, 0, 0)),
            pl.BlockSpec((1, _BG, 128), lambda i: (i, 0, 0)),
            pl.BlockSpec((D, D), lambda i: (0, 0)),
            pl.BlockSpec((1, D), lambda i: (0, 0)),
        ],
        out_specs=pl.BlockSpec((_BM, D), lambda i: (i, 0)),
        out_shape=jax.ShapeDtypeStruct((N_PAD, D), jnp.float32),
    )(parts4d, dinv2d, wt, b2)


def kernel(x, edge_index, x0, W, b):
    e3d = edge_index.astype(jnp.int32).reshape(NC, NCH, EC)
    parts, _, dinv = _sc_kernel(x, e3d)
    out = _matmul(parts.reshape(NC, N_PAD // 128, 128, D),
                  dinv.reshape(N_PAD // _BM, _BG, 128), W.T, b.reshape(1, D))
    return out[:N_NODES]


# speedup vs baseline: 3.4030x; 1.0137x over previous
import jax
import jax.numpy as jnp
from jax import lax
from jax.experimental import pallas as pl
from jax.experimental.pallas import tpu as pltpu
from jax.experimental.pallas import tpu_sc as plsc

N_NODES = 10000
N_EDGES = 320000
D = 128

NC = 2
NS = 16
L = 16

N_PAD = 10240
ROWS_PER_TILE = N_PAD // NS
GROUPS_PER_TILE = ROWS_PER_TILE // L
RC = 80
N_RCHUNK = ROWS_PER_TILE // RC

EC = 128
NCH = N_EDGES // EC
SLAB = 40
N_SLAB = NCH // SLAB
TAIL = NCH - N_SLAB * SLAB


def _rsqrt16(v):
    i = lax.bitcast_convert_type(v, jnp.int32)
    i = jnp.int32(0x5F3759DF) - lax.shift_right_logical(i, 1)
    y = lax.bitcast_convert_type(i, jnp.float32)
    half = v * 0.5
    for _ in range(3):
        y = y * (1.5 - half * y * y)
    return y


def _scale_rows(vb, dv, base):
    for g in range(RC // L):
        dvv = dv[pl.ds(base + g * L, L)]
        for r in range(L):
            bc = lax.gather(
                dvv, jnp.full((L, 1), r, jnp.int32),
                lax.GatherDimensionNumbers(offset_dims=(),
                                           collapsed_slice_dims=(0,),
                                           start_index_map=(0,)),
                slice_sizes=(1,),
                mode=lax.GatherScatterMode.PROMISE_IN_BOUNDS)
            row = g * L + r
            for j in range(D // L):
                vb[row, pl.ds(j * L, L)] = vb[row, pl.ds(j * L, L)] * bc


def _sc_body(x_hbm, e3d_hbm,
             part_hbm, xs_hbm, dinv_hbm,
             vb0, vb1, rb2d, cb2d,
             ones, dv,
             deg_sh, agg_sh,
             semA, semB, semC, semD, semE, semF):
    c = lax.axis_index("c")
    s = lax.axis_index("s")
    row0 = s * ROWS_PER_TILE

    def p3_valid(kk):
        return row0 + kk * RC + RC <= N_NODES

    def p3_load(kk, vb, sem):
        return pltpu.async_copy(
            x_hbm.at[pl.ds(row0 + kk * RC, RC)], vb.at[pl.ds(0, RC)], sem)

    with jax.named_scope("p0_init"):
        for g in range(EC // L):
            ones[pl.ds(g * L, L)] = jnp.full((L,), 1.0, jnp.float32)

        def zvb(r, carry):
            for j in range(D // L):
                vb0[r, pl.ds(j * L, L)] = jnp.zeros((L,), jnp.float32)
            return carry
        lax.fori_loop(0, EC, zvb, 0)

        def zdv(k, carry):
            dv[pl.ds(k * L, L)] = jnp.zeros((L,), jnp.float32)
            return carry
        lax.fori_loop(0, GROUPS_PER_TILE, zdv, 0)
        pltpu.sync_copy(dv, deg_sh.at[pl.ds(row0, ROWS_PER_TILE)])
        for g in range(ROWS_PER_TILE // EC):
            pltpu.sync_copy(vb0, agg_sh.at[pl.ds(row0 + g * EC, EC)])

        @pl.when(p3_valid(0))
        def _():
            p3_load(0, vb0, semE)

        @pl.when(p3_valid(1))
        def _():
            p3_load(1, vb1, semF)
        plsc.subcore_barrier()

    with jax.named_scope("p1_degree"):
        sems = (semA, semB, semC, semD)

        def dsc_start(j, sem):
            pltpu.async_copy(ones, deg_sh.at[rb2d.at[j]], sem, add=True)

        def dsc_wait(j, sem):
            pltpu.make_async_copy(ones, deg_sh.at[rb2d.at[j]], sem).wait()

        def hist_run(chunk0, n):
            pltpu.sync_copy(e3d_hbm.at[1, pl.ds(chunk0, n)],
                            rb2d.at[pl.ds(0, n)])
            for j in range(4):
                dsc_start(j, sems[j])

            def step(k, carry):
                a = 4 * k
                for q in range(4):
                    dsc_wait(a + q, sems[q])

                    @pl.when(a + 4 + q < n)
                    def _(q=q):
                        dsc_start(a + 4 + q, sems[q])
                return carry
            lax.fori_loop(0, n // 4, step, 0)

        for hh in range(4):
            @pl.when(s + 16 * hh < N_SLAB)
            def _():
                hist_run((s + 16 * hh) * SLAB, SLAB)

        @pl.when(s == NS - 1)
        def _():
            hist_run(N_SLAB * SLAB, TAIL)
        plsc.subcore_barrier()

    with jax.named_scope("p2_dinv"):
        pltpu.sync_copy(deg_sh.at[pl.ds(row0, ROWS_PER_TILE)], dv)

        def dinv_step(k, carry):
            v = dv[pl.ds(k * L, L)]
            y = jnp.where(v >= 0.5, _rsqrt16(v), 0.0)
            dv[pl.ds(k * L, L)] = y
            return carry
        lax.fori_loop(0, GROUPS_PER_TILE, dinv_step, 0)

        @pl.when(c == 0)
        def _():
            pltpu.sync_copy(dv, dinv_hbm.at[pl.ds(row0, ROWS_PER_TILE)])

    with jax.named_scope("p3_xs"):
        def p3_do(kk, vb, sem):
            @pl.when(p3_valid(kk))
            def _():
                pltpu.make_async_copy(
                    x_hbm.at[pl.ds(row0 + kk * RC, RC)],
                    vb.at[pl.ds(0, RC)], sem).wait()
                _scale_rows(vb, dv, kk * RC)
                pltpu.sync_copy(vb.at[pl.ds(0, RC)],
                                xs_hbm.at[pl.ds(row0 + kk * RC, RC)])

        def p3_step(k2, carry):
            a = 2 * k2
            p3_do(a, vb0, semE)

            @pl.when((a + 2 < N_RCHUNK) & p3_valid(a + 2))
            def _():
                p3_load(a + 2, vb0, semE)
            p3_do(a + 1, vb1, semF)

            @pl.when((a + 3 < N_RCHUNK) & p3_valid(a + 3))
            def _():
                p3_load(a + 3, vb1, semF)
            return carry
        lax.fori_loop(0, N_RCHUNK // 2, p3_step, 0)
        plsc.subcore_barrier()

    with jax.named_scope("p4_edges"):
        def g_start(j, vb, sem):
            return pltpu.async_copy(xs_hbm.at[rb2d.at[j]], vb, sem)

        def g_wait(j, vb, sem):
            pltpu.make_async_copy(xs_hbm.at[rb2d.at[j]], vb, sem).wait()

        def edge_run(chunk0, n):
            pltpu.sync_copy(e3d_hbm.at[0, pl.ds(chunk0, n)],
                            rb2d.at[pl.ds(0, n)])
            pltpu.sync_copy(e3d_hbm.at[1, pl.ds(chunk0, n)],
                            cb2d.at[pl.ds(0, n)])
            g_start(0, vb0, semA)

            def step(k, carry):
                a = 2 * k
                g_start(a + 1, vb1, semB)
                g_wait(a, vb0, semA)
                pltpu.sync_copy(vb0, agg_sh.at[cb2d.at[a]], add=True)

                @pl.when(a + 2 < n)
                def _():
                    g_start(a + 2, vb0, semA)
                g_wait(a + 1, vb1, semB)
                pltpu.sync_copy(vb1, agg_sh.at[cb2d.at[a + 1]], add=True)
                return carry
            lax.fori_loop(0, n // 2, step, 0)

        for hh in range(2):
            @pl.when(s + 16 * hh < N_SLAB // 2)
            def _():
                edge_run((2 * (s + 16 * hh) + c) * SLAB, SLAB)

        @pl.when((s == NS - 1) & (c == 0))
        def _():
            edge_run(N_SLAB * SLAB, TAIL)
        plsc.subcore_barrier()

    with jax.named_scope("p5_out"):
        for g in range(ROWS_PER_TILE // EC):
            vb = vb0 if g % 2 == 0 else vb1
            sem = semA if g % 2 == 0 else semB
            if g >= 2:
                pltpu.make_async_copy(
                    vb, part_hbm.at[c, pl.ds(row0 + (g - 2) * EC, EC)],
                    sem).wait()
            pltpu.sync_copy(agg_sh.at[pl.ds(row0 + g * EC, EC)], vb)
            pltpu.async_copy(vb, part_hbm.at[c, pl.ds(row0 + g * EC, EC)],
                             sem)
        for g in (3, 4):
            vb = vb0 if g % 2 == 0 else vb1
            sem = semA if g % 2 == 0 else semB
            pltpu.make_async_copy(
                vb, part_hbm.at[c, pl.ds(row0 + g * EC, EC)], sem).wait()


_sc_kernel = pl.kernel(
    _sc_body,
    out_type=[
        jax.ShapeDtypeStruct((NC, N_PAD, D), jnp.float32),
        jax.ShapeDtypeStruct((N_NODES, D), jnp.float32),
        jax.ShapeDtypeStruct((N_PAD,), jnp.float32),
    ],
    mesh=plsc.VectorSubcoreMesh(core_axis_name="c", subcore_axis_name="s"),
    compiler_params=pltpu.CompilerParams(needs_layout_passes=False),
    scratch_types=[
        pltpu.VMEM((EC, D), jnp.float32),
        pltpu.VMEM((EC, D), jnp.float32),
        pltpu.VMEM((SLAB, EC), jnp.int32),
        pltpu.VMEM((SLAB, EC), jnp.int32),
        pltpu.VMEM((EC,), jnp.float32),
        pltpu.VMEM((ROWS_PER_TILE,), jnp.float32),
        pltpu.VMEM_SHARED((N_PAD,), jnp.float32),
        pltpu.VMEM_SHARED((N_PAD, D), jnp.float32),
        pltpu.SemaphoreType.DMA,
        pltpu.SemaphoreType.DMA,
        pltpu.SemaphoreType.DMA,
        pltpu.SemaphoreType.DMA,
        pltpu.SemaphoreType.DMA,
        pltpu.SemaphoreType.DMA,
    ],
)


def _mm_body(p_ref, dv_ref, wt_ref, b_ref, o_ref):
    a = (p_ref[0] + p_ref[1]) * dv_ref[0][:, :, None]
    a = a.reshape(a.shape[0] * a.shape[1], a.shape[2])
    o_ref[...] = (
        jnp.dot(a, wt_ref[...], preferred_element_type=jnp.float32) + b_ref[...]
    )


_BM = 2048
_BG = _BM // 128


def _matmul(parts4d, dinv2d, wt, b2):
    return pl.pallas_call(
        _mm_body,
        grid=(N_PAD // _BM,),
        in_specs=[
            pl.BlockSpec((NC, _BG, 128, D), lambda i: (0, i, 0, 0)),
            pl.BlockSpec((1, _BG, 128), lambda i: (i, 0, 0)),
            pl.BlockSpec((D, D), lambda i: (0, 0)),
            pl.BlockSpec((1, D), lambda i: (0, 0)),
        ],
        out_specs=pl.BlockSpec((_BM, D), lambda i: (i, 0)),
        out_shape=jax.ShapeDtypeStruct((N_PAD, D), jnp.float32),
    )(parts4d, dinv2d, wt, b2)


def kernel(x, edge_index, x0, W, b):
    e3d = edge_index.astype(jnp.int32).reshape(NC, NCH, EC)
    parts, _, dinv = _sc_kernel(x, e3d)
    out = _matmul(parts.reshape(NC, N_PAD // 128, 128, D),
                  dinv.reshape(N_PAD // _BM, _BG, 128), W.T, b.reshape(1, D))
    return out[:N_NODES]
